# baseline (speedup 1.0000x reference)
import functools
import math

import jax
import jax.numpy as jnp
from jax import lax
from jax.experimental import pallas as pl
from jax.experimental.pallas import tpu as pltpu
from jax.experimental.pallas import tpu_sc as plsc

D_MODEL = 64
SCALE = math.sqrt(D_MODEL)

NUM_CORES = 2
NUM_SUBCORES = 16
NUM_WORKERS = NUM_CORES * NUM_SUBCORES
LANES = 16

CHUNK = 128


def _emb_body(idx_hbm, table_hbm, out_hbm, idx_v, rows_v, sem):
    per_w = idx_v.shape[0]
    wid = lax.axis_index("s") * NUM_CORES + lax.axis_index("c")
    base_chunk = wid * per_w
    pltpu.sync_copy(idx_hbm.at[pl.ds(base_chunk, per_w)], idx_v)

    def chunk_body(j, carry):
        pltpu.async_copy(table_hbm.at[idx_v.at[j]], rows_v, sem).wait()

        def scale_row(i, c):
            for t in range(D_MODEL // LANES):
                sl = pl.ds(t * LANES, LANES)
                rows_v[i, sl] = rows_v[i, sl] * SCALE
            return c

        lax.fori_loop(0, CHUNK, scale_row, 0)
        pltpu.sync_copy(rows_v, out_hbm.at[pl.ds((base_chunk + j) * CHUNK, CHUNK)])
        return carry

    lax.fori_loop(0, per_w, chunk_body, 0)


def kernel(x, lut_weight):
    b0, b1 = x.shape
    total = b0 * b1
    n_chunks = total // CHUNK
    per_w = n_chunks // NUM_WORKERS
    idx2d = x.reshape(n_chunks, CHUNK).astype(jnp.int32)

    mesh = plsc.VectorSubcoreMesh(core_axis_name="c", subcore_axis_name="s")
    emb = functools.partial(
        pl.kernel,
        mesh=mesh,
        out_type=jax.ShapeDtypeStruct((total, D_MODEL), jnp.float32),
        scratch_types=[
            pltpu.VMEM((per_w, CHUNK), jnp.int32),
            pltpu.VMEM((CHUNK, D_MODEL), jnp.float32),
            pltpu.SemaphoreType.DMA,
        ],
        compiler_params=pltpu.CompilerParams(use_tc_tiling_on_sc=False),
    )(_emb_body)
    out = emb(idx2d, lut_weight)
    return out.reshape(b0, b1, D_MODEL)


# speedup vs baseline: 1.2067x; 1.2067x over previous
import functools
import math

import jax
import jax.numpy as jnp
from jax import lax
from jax.experimental import pallas as pl
from jax.experimental.pallas import tpu as pltpu
from jax.experimental.pallas import tpu_sc as plsc

D_MODEL = 64
SCALE = math.sqrt(D_MODEL)

NUM_CORES = 2
NUM_SUBCORES = 16
NUM_WORKERS = NUM_CORES * NUM_SUBCORES
LANES = 16

CHUNK = 128
NBUF = 4


def _emb_body(idx_hbm, table_hbm, out_hbm, idx_v, gbuf, gsem, wsem):
    per_w = idx_v.shape[0]
    wid = lax.axis_index("s") * NUM_CORES + lax.axis_index("c")
    base_chunk = wid * per_w
    pltpu.sync_copy(idx_hbm.at[pl.ds(base_chunk, per_w)], idx_v)

    def gather(c, b):
        return pltpu.make_async_copy(
            table_hbm.at[idx_v.at[c]], gbuf.at[b], gsem.at[b])

    def writeback(c, b):
        return pltpu.make_async_copy(
            gbuf.at[b], out_hbm.at[pl.ds((base_chunk + c) * CHUNK, CHUNK)],
            wsem.at[b])

    for b in range(NBUF):
        gather(b, b).start()

    n_groups = per_w // NBUF

    def group(it, carry):
        for b in range(NBUF):
            c = it * NBUF + b
            gather(c, b).wait()

            def scale_row(i, _):
                for t in range(D_MODEL // LANES):
                    sl = pl.ds(t * LANES, LANES)
                    gbuf[b, i, sl] = gbuf[b, i, sl] * SCALE
                return 0

            lax.fori_loop(0, CHUNK, scale_row, 0, unroll=2)
            writeback(c, b).start()

            @pl.when(it < n_groups - 1)
            def _refill():
                writeback(c, b).wait()
                gather(c + NBUF, b).start()

        return carry

    lax.fori_loop(0, n_groups, group, 0)
    for b in range(NBUF):
        writeback(per_w - NBUF + b, b).wait()


def kernel(x, lut_weight):
    b0, b1 = x.shape
    total = b0 * b1
    n_chunks = total // CHUNK
    per_w = n_chunks // NUM_WORKERS
    idx2d = x.reshape(n_chunks, CHUNK).astype(jnp.int32)

    mesh = plsc.VectorSubcoreMesh(core_axis_name="c", subcore_axis_name="s")
    emb = functools.partial(
        pl.kernel,
        mesh=mesh,
        out_type=jax.ShapeDtypeStruct((total, D_MODEL), jnp.float32),
        scratch_types=[
            pltpu.VMEM((per_w, CHUNK), jnp.int32),
            pltpu.VMEM((NBUF, CHUNK, D_MODEL), jnp.float32),
            pltpu.SemaphoreType.DMA((NBUF,)),
            pltpu.SemaphoreType.DMA((NBUF,)),
        ],
        compiler_params=pltpu.CompilerParams(use_tc_tiling_on_sc=False),
    )(_emb_body)
    out = emb(idx2d, lut_weight)
    return out.reshape(b0, b1, D_MODEL)
